# baseline (speedup 1.0000x reference)
import functools

import jax
import jax.numpy as jnp
from jax import lax
from jax.experimental import pallas as pl
from jax.experimental.pallas import tpu as pltpu
from jax.experimental.pallas import tpu_sc as plsc

D = 256
NKX = D // 2 + 1
NIMG = 32
NPTS = NIMG * D * NKX
NVOX = D * D * NKX
HS = 512
HSZ = NVOX // HS
NW = 32
CHUNK = NPTS // NW
LANES = 16
NBINS = HS * LANES
SEG_PAD = HS * LANES
TRASH = NPTS + SEG_PAD
TRASH_SZ = 1 << 20
NBLEN = TRASH + TRASH_SZ + 4096

HBLK = 2064
RBLK = 768
ABLK = 2048


_SC_PARAMS = pltpu.CompilerParams(needs_layout_passes=False)


def _mesh():
    return plsc.VectorSubcoreMesh(core_axis_name="c", subcore_axis_name="s")


def _wid():
    return lax.axis_index("s") * 2 + lax.axis_index("c")


def _prep_body(fr_ref, fi_ref, ctf_ref, neg_ref, syky_ref, sxkx_ref,
               vr_ref, vi_ref, cc_ref):
    fr = fr_ref[0]
    fi = fi_ref[0]
    ctf = ctf_ref[0]
    ph = (-2.0 * jnp.pi) * (syky_ref[0, 0][:, None] + sxkx_ref[0, 0][None, :])
    c = jnp.cos(ph)
    s = jnp.sin(ph)
    pr = (fr * c - fi * s) * ctf
    pi = (fr * s + fi * c) * ctf
    sign = 1.0 - 2.0 * neg_ref[0]
    vr_ref[0] = pr
    vi_ref[0] = pi * sign
    cc_ref[0] = ctf * ctf


def _prep(fr, fi, ctf, negf, syky, sxkx):
    blk = pl.BlockSpec((1, D, NKX), lambda b: (b, 0, 0))
    sblk = lambda n: pl.BlockSpec((1, 1, n), lambda b: (b, 0, 0))
    return pl.pallas_call(
        _prep_body,
        grid=(NIMG,),
        in_specs=[blk, blk, blk, blk, sblk(D), sblk(NKX)],
        out_specs=[blk, blk, blk],
        out_shape=[jax.ShapeDtypeStruct((NIMG, D, NKX), jnp.float32)] * 3,
    )(fr, fi, ctf, negf, syky.reshape(NIMG, 1, D), sxkx.reshape(NIMG, 1, NKX))


def _hist_body(idxf, table, ibuf, hist, sem):
    wid = _wid()
    lane = lax.iota(jnp.int32, LANES)
    zero16 = jnp.zeros((LANES,), jnp.int32)
    one16 = jnp.ones((LANES,), jnp.int32)

    def zero(i, carry):
        hist[pl.ds(i * 16, 16)] = zero16
        return carry

    lax.fori_loop(0, NBINS // 16, zero, 0)

    def blk(j, carry):
        pltpu.sync_copy(idxf.at[pl.ds(wid * CHUNK + j * HBLK, HBLK)], ibuf)
        for v in range(HBLK // 16):
            iv = ibuf[pl.ds(v * 16, 16)]
            m = iv < NVOX
            binc = lax.div(iv, HSZ) * LANES + lane
            plsc.addupdate_scatter(hist, [binc], one16, mask=m)
        return carry

    lax.fori_loop(0, CHUNK // HBLK, blk, 0)
    pltpu.sync_copy(hist, table.at[wid])


def _hist(idxf):
    k = functools.partial(
        pl.kernel,
        mesh=_mesh(),
        compiler_params=_SC_PARAMS,
        out_type=jax.ShapeDtypeStruct((NW, NBINS), jnp.int32),
        scratch_types=[
            pltpu.VMEM((HBLK,), jnp.int32),
            pltpu.VMEM((NBINS,), jnp.int32),
            pltpu.SemaphoreType.DMA,
        ],
    )
    return k(_hist_body)(idxf)


def _reorder_body(idxf, vr, vi, cc, base2, bidx, bvr, bvi, bcc,
                  nxt, ibuf, rbuf, vrbuf, vibuf, ccbuf, dbuf, sem):
    wid = _wid()
    lane = lax.iota(jnp.int32, LANES)
    pltpu.sync_copy(base2.at[wid], nxt)

    def blk(j, carry):
        off = wid * CHUNK + j * RBLK
        pltpu.sync_copy(idxf.at[pl.ds(off, RBLK)], ibuf)
        pltpu.sync_copy(vr.at[pl.ds(off, RBLK)], vrbuf)
        pltpu.sync_copy(vi.at[pl.ds(off, RBLK)], vibuf)
        pltpu.sync_copy(cc.at[pl.ds(off, RBLK)], ccbuf)
        for v in range(RBLK // 16):
            iv = ibuf[pl.ds(v * 16, 16)]
            m = iv < NVOX
            hs = lax.div(iv, HSZ)
            binc = hs * LANES + lane
            cur = plsc.load_gather(nxt, [binc], mask=m)
            plsc.store_scatter(nxt, [binc], cur + 1, mask=m)
            rbuf[pl.ds(v * 16, 16)] = iv - hs * HSZ
            tr = TRASH + ((wid * CHUNK + j * RBLK + v * 16 + lane) & (TRASH_SZ - 1))
            dest = jnp.where(m, cur, tr)
            dbuf[v // 8, pl.ds((v % 8) * 16, 16)] = dest
        handles = []
        for q in range(RBLK // 128):
            idx_ref = dbuf.at[q]
            sl = pl.ds(q * 128, 128)
            handles.append(pltpu.async_copy(rbuf.at[sl], bidx.at[idx_ref], sem))
            handles.append(pltpu.async_copy(vrbuf.at[sl], bvr.at[idx_ref], sem))
            handles.append(pltpu.async_copy(vibuf.at[sl], bvi.at[idx_ref], sem))
            handles.append(pltpu.async_copy(ccbuf.at[sl], bcc.at[idx_ref], sem))
        for h in handles:
            h.wait()
        return carry

    lax.fori_loop(0, CHUNK // RBLK, blk, 0)


def _reorder(idxf, vr, vi, cc, base2):
    k = functools.partial(
        pl.kernel,
        mesh=_mesh(),
        compiler_params=_SC_PARAMS,
        out_type=[
            jax.ShapeDtypeStruct((NBLEN,), jnp.int32),
            jax.ShapeDtypeStruct((NBLEN,), jnp.float32),
            jax.ShapeDtypeStruct((NBLEN,), jnp.float32),
            jax.ShapeDtypeStruct((NBLEN,), jnp.float32),
        ],
        scratch_types=[
            pltpu.VMEM((NBINS,), jnp.int32),
            pltpu.VMEM((RBLK,), jnp.int32),
            pltpu.VMEM((RBLK,), jnp.int32),
            pltpu.VMEM((RBLK,), jnp.float32),
            pltpu.VMEM((RBLK,), jnp.float32),
            pltpu.VMEM((RBLK,), jnp.float32),
            pltpu.VMEM((RBLK // 128, 128), jnp.int32),
            pltpu.SemaphoreType.DMA,
        ],
    )
    return k(_reorder_body)(idxf, vr, vi, cc, base2)


def _accum_body(bidx, bvr, bvi, bcc, starts, lens, numflat, wts, csq,
                acr, aci, acw, acc, rbuf, vrbuf, vibuf, ccbuf, sbuf, lbuf,
                sem, semw):
    wid = _wid()
    lane = lax.iota(jnp.int32, LANES)
    zerof = jnp.zeros((LANES,), jnp.float32)
    onef = jnp.ones((LANES,), jnp.float32)
    pltpu.sync_copy(starts, sbuf)
    pltpu.sync_copy(lens, lbuf)

    def zero_one(ac):
        def zero(i, c2):
            ac[pl.ds(i * 16, 16)] = zerof
            return c2
        lax.fori_loop(0, HSZ // 16, zero, 0)

    for ac in (acr, aci, acw, acc):
        zero_one(ac)

    def rnd(r, carry):
        hs = r * NW + wid
        hsv = jnp.full((LANES,), hs, jnp.int32)
        start = jnp.max(plsc.load_gather(sbuf, [hsv]))
        seglen = jnp.max(plsc.load_gather(lbuf, [hsv]))

        def blk(j, c2):
            gpos = pl.multiple_of(start + j * ABLK, 16)
            pltpu.sync_copy(bidx.at[pl.ds(gpos, ABLK)], rbuf)
            pltpu.sync_copy(bvr.at[pl.ds(gpos, ABLK)], vrbuf)
            pltpu.sync_copy(bvi.at[pl.ds(gpos, ABLK)], vibuf)
            pltpu.sync_copy(bcc.at[pl.ds(gpos, ABLK)], ccbuf)
            lim = jnp.full((LANES,), seglen - j * ABLK, jnp.int32)
            for v in range(ABLK // 16):
                m = (v * 16 + lane) < lim
                rel = rbuf[pl.ds(v * 16, 16)]
                plsc.addupdate_scatter(acr, [rel], vrbuf[pl.ds(v * 16, 16)], mask=m)
                plsc.addupdate_scatter(aci, [rel], vibuf[pl.ds(v * 16, 16)], mask=m)
                plsc.addupdate_scatter(acw, [rel], onef, mask=m)
                plsc.addupdate_scatter(acc, [rel], ccbuf[pl.ds(v * 16, 16)], mask=m)
            return c2

        nblk = lax.div(seglen + (ABLK - 1), ABLK)
        lax.fori_loop(0, nblk, blk, 0)
        hbase = hs * HSZ
        h0 = pltpu.async_copy(acr, numflat.at[0, pl.ds(hbase, HSZ)], semw)
        h1 = pltpu.async_copy(aci, numflat.at[1, pl.ds(hbase, HSZ)], semw)
        h2 = pltpu.async_copy(acw, wts.at[pl.ds(hbase, HSZ)], semw)
        h3 = pltpu.async_copy(acc, csq.at[pl.ds(hbase, HSZ)], semw)
        for h, ac in ((h0, acr), (h1, aci), (h2, acw), (h3, acc)):
            h.wait()
            zero_one(ac)
        return carry

    lax.fori_loop(0, HS // NW, rnd, 0)


def _accum(bidx, bvr, bvi, bcc, starts, lens):
    k = functools.partial(
        pl.kernel,
        mesh=_mesh(),
        compiler_params=_SC_PARAMS,
        out_type=[
            jax.ShapeDtypeStruct((2, NVOX), jnp.float32),
            jax.ShapeDtypeStruct((NVOX,), jnp.float32),
            jax.ShapeDtypeStruct((NVOX,), jnp.float32),
        ],
        scratch_types=[
            pltpu.VMEM((HSZ,), jnp.float32),
            pltpu.VMEM((HSZ,), jnp.float32),
            pltpu.VMEM((HSZ,), jnp.float32),
            pltpu.VMEM((HSZ,), jnp.float32),
            pltpu.VMEM((ABLK,), jnp.int32),
            pltpu.VMEM((ABLK,), jnp.float32),
            pltpu.VMEM((ABLK,), jnp.float32),
            pltpu.VMEM((ABLK,), jnp.float32),
            pltpu.VMEM((520,), jnp.int32),
            pltpu.VMEM((520,), jnp.int32),
            pltpu.SemaphoreType.DMA,
            pltpu.SemaphoreType.DMA,
        ],
    )
    return k(_accum_body)(bidx, bvr, bvi, bcc, starts, lens)


def kernel(imgs, ctf, rotMats, hwShiftAngs, numerator, weights, ctfsq):
    f = jnp.fft.rfftn(imgs, axes=(-2, -1))
    fr = jnp.real(f).astype(jnp.float32)
    fi = jnp.imag(f).astype(jnp.float32)
    ky = jnp.fft.fftfreq(D).astype(jnp.float32)
    kx = jnp.fft.rfftfreq(D).astype(jnp.float32)
    syky = hwShiftAngs[:, 0, None] * ky[None, :]
    sxkx = hwShiftAngs[:, 1, None] * kx[None, :]

    yc = (jnp.fft.fftfreq(D) * D).astype(jnp.float32)
    xc = jnp.arange(NKX, dtype=jnp.float32)
    gx = jnp.broadcast_to(xc[None, :], (D, NKX))
    gy = jnp.broadcast_to(yc[:, None], (D, NKX))
    gz = jnp.zeros((D, NKX), dtype=jnp.float32)
    grid = jnp.stack([gx, gy, gz], axis=-1)
    rot = jnp.einsum('bij,hwj->bhwi', rotMats, grid)
    neg = rot[..., 0] < 0
    rot = jnp.where(neg[..., None], -rot, rot)
    xi = jnp.round(rot[..., 0]).astype(jnp.int32)
    yi = jnp.round(rot[..., 1]).astype(jnp.int32)
    zi = jnp.round(rot[..., 2]).astype(jnp.int32)
    half = D // 2
    valid = (xi >= 0) & (xi < NKX) & (jnp.abs(yi) < half) & (jnp.abs(zi) < half)
    yi = jnp.mod(yi, D)
    zi = jnp.mod(zi, D)
    flat = (zi * D + yi) * NKX + xi
    idxf = jnp.where(valid, flat, NVOX).reshape(NPTS)

    vr, vi, cc = _prep(fr, fi, ctf, neg.astype(jnp.float32), syky, sxkx)
    vr = vr.reshape(NPTS)
    vi = vi.reshape(NPTS)
    cc = cc.reshape(NPTS)

    table = _hist(idxf)

    t = table.reshape(NW, HS, LANES).transpose(1, 0, 2)
    tot = t.sum(axis=(1, 2))
    ptot = ((tot + 15) // 16) * 16
    seg_start = jnp.concatenate([jnp.zeros((1,), jnp.int32),
                                 jnp.cumsum(ptot)[:-1].astype(jnp.int32)])
    tf = t.reshape(HS, NW * LANES)
    inner = jnp.cumsum(tf, axis=1).astype(jnp.int32) - tf
    base = seg_start[:, None] + inner
    base2 = (base.reshape(HS, NW, LANES).transpose(1, 0, 2)
             .reshape(NW, NBINS).astype(jnp.int32))
    starts = jnp.zeros((520,), jnp.int32).at[:HS].set(seg_start)
    lens = jnp.zeros((520,), jnp.int32).at[:HS].set(tot.astype(jnp.int32))

    bidx, bvr, bvi, bcc = _reorder(idxf, vr, vi, cc, base2)
    numflat, wts, csq = _accum(bidx, bvr, bvi, bcc, starts, lens)

    new_num = numflat.reshape(2, D, D, NKX)
    new_w = wts.reshape(D, D, NKX)
    new_c = csq.reshape(D, D, NKX)
    return new_num, new_w, new_c


# speedup vs baseline: 4.4573x; 4.4573x over previous
import functools

import jax
import jax.numpy as jnp
from jax import lax
from jax.experimental import pallas as pl
from jax.experimental.pallas import tpu as pltpu
from jax.experimental.pallas import tpu_sc as plsc

D = 256
NKX = D // 2 + 1
NIMG = 32
NPTS = NIMG * D * NKX
NZY = D * D
SENT = 1 << 24
QS = 1024
QROWS = 64
NW = 32
CHUNK = NPTS // NW
SUB = 4
SCH = CHUNK // SUB
LANES = 16
LBINS = (QS + 1) * LANES
LBINS_P = 16416
RS = SCH + QS + 16
TBLN = 1040
BRUN = 64

REC_W = 4
REGW = RS * REC_W
BRECW = NW * SUB * REGW + 1024

_SC_PARAMS = pltpu.CompilerParams(needs_layout_passes=False)


def _mesh():
    return plsc.VectorSubcoreMesh(core_axis_name="c", subcore_axis_name="s")


def _wid():
    return lax.axis_index("s") * 2 + lax.axis_index("c")


def _prep_body(fr_ref, fi_ref, ctf_ref, neg_ref, syky_ref, sxkx_ref,
               vr_ref, vi_ref, cc_ref):
    fr = fr_ref[0]
    fi = fi_ref[0]
    ctf = ctf_ref[0]
    ph = (-2.0 * jnp.pi) * (syky_ref[0, 0][:, None] + sxkx_ref[0, 0][None, :])
    c = jnp.cos(ph)
    s = jnp.sin(ph)
    pr = (fr * c - fi * s) * ctf
    pi = (fr * s + fi * c) * ctf
    sign = 1.0 - 2.0 * neg_ref[0]
    vr_ref[0] = pr
    vi_ref[0] = pi * sign
    cc_ref[0] = ctf * ctf


def _prep(fr, fi, ctf, negf, syky, sxkx):
    blk = pl.BlockSpec((1, D, NKX), lambda b: (b, 0, 0))
    sblk = lambda n: pl.BlockSpec((1, 1, n), lambda b: (b, 0, 0))
    return pl.pallas_call(
        _prep_body,
        grid=(NIMG,),
        in_specs=[blk, blk, blk, blk, sblk(D), sblk(NKX)],
        out_specs=[blk, blk, blk],
        out_shape=[jax.ShapeDtypeStruct((NIMG, D, NKX), jnp.float32)] * 3,
    )(fr, fi, ctf, negf, syky.reshape(NIMG, 1, D), sxkx.reshape(NIMG, 1, NKX))


def _sortflush_body(idxf, vr, vi, cc, brec, offt, lent,
                    ibuf, vbr, vbi, vbc, lhist, loff, ptab, ltab,
                    sorted_buf, sem):
    wid = _wid()
    lane = lax.iota(jnp.int32, LANES)
    zero16 = jnp.zeros((LANES,), jnp.int32)
    one16 = jnp.ones((LANES,), jnp.int32)

    def subchunk(s, carry):
        base = wid * CHUNK + s * SCH
        region = (wid * SUB + s) * RS

        pltpu.sync_copy(idxf.at[pl.ds(base, SCH)], ibuf)
        pltpu.sync_copy(vr.at[pl.ds(base, SCH)], vbr)
        pltpu.sync_copy(vi.at[pl.ds(base, SCH)], vbi)
        pltpu.sync_copy(cc.at[pl.ds(base, SCH)], vbc)

        def zero(i, c2):
            lhist[pl.ds(i * 16, 16)] = zero16
            return c2

        lax.fori_loop(0, LBINS_P // 16, zero, 0)

        def zero2(i, c2):
            ltab[pl.ds(i * 16, 16)] = zero16
            return c2

        lax.fori_loop(0, TBLN // 16, zero2, 0)

        def h1(v, c2):
            iv = ibuf[pl.ds(v * 16, 16)]
            q = lax.shift_right_logical(iv, 14)
            plsc.addupdate_scatter(lhist, [q * LANES + lane], one16)
            plsc.addupdate_scatter(ltab, [q], one16)
            return c2

        lax.fori_loop(0, SCH // 16, h1, 0)

        def pfxa(g, run):
            tot = ltab[pl.ds(g * 16, 16)]
            ptot = (tot + 1) & jnp.int32(~1)
            cs = plsc.cumsum(ptot)
            ptab[pl.ds(g * 16, 16)] = (cs - ptot) + jnp.full(
                (LANES,), run + region, jnp.int32)
            return run + jnp.max(cs)

        lax.fori_loop(0, (QS + 16) // 16, pfxa, jnp.int32(0))

        pltpu.sync_copy(ptab, offt.at[wid, s])
        pltpu.sync_copy(ltab, lent.at[wid, s])

        def pfxb(q, c2):
            h = lhist[pl.ds(q * 16, 16)]
            excl = plsc.cumsum(h) - h
            qb = plsc.load_gather(ptab, [jnp.full((LANES,), q, jnp.int32)])
            loff[pl.ds(q * 16, 16)] = excl + qb - jnp.full(
                (LANES,), region, jnp.int32)
            return c2

        lax.fori_loop(0, QS + 1, pfxb, 0)

        def p2(v, c2):
            iv = ibuf[pl.ds(v * 16, 16)]
            binc = lax.shift_right_logical(iv, 14) * LANES + lane
            pos = plsc.load_gather(loff, [binc])
            plsc.store_scatter(loff, [binc], pos + 1)
            widx = pos * REC_W
            rel = iv & jnp.int32(16383)
            plsc.store_scatter(sorted_buf, [widx],
                               plsc.bitcast(rel, jnp.float32))
            plsc.store_scatter(sorted_buf, [widx + 1], vbr[pl.ds(v * 16, 16)])
            plsc.store_scatter(sorted_buf, [widx + 2], vbi[pl.ds(v * 16, 16)])
            plsc.store_scatter(sorted_buf, [widx + 3], vbc[pl.ds(v * 16, 16)])
            return c2

        lax.fori_loop(0, SCH // 16, p2, 0)

        pltpu.sync_copy(sorted_buf, brec.at[pl.ds(region * REC_W, REGW)])
        return carry

    lax.fori_loop(0, SUB, subchunk, 0)


def _sortflush(idxf, vr, vi, cc):
    k = functools.partial(
        pl.kernel,
        mesh=_mesh(),
        compiler_params=_SC_PARAMS,
        out_type=[
            jax.ShapeDtypeStruct((BRECW,), jnp.float32),
            jax.ShapeDtypeStruct((NW, SUB, TBLN), jnp.int32),
            jax.ShapeDtypeStruct((NW, SUB, TBLN), jnp.int32),
        ],
        scratch_types=[
            pltpu.VMEM((SCH,), jnp.int32),
            pltpu.VMEM((SCH,), jnp.float32),
            pltpu.VMEM((SCH,), jnp.float32),
            pltpu.VMEM((SCH,), jnp.float32),
            pltpu.VMEM((LBINS_P,), jnp.int32),
            pltpu.VMEM((LBINS_P,), jnp.int32),
            pltpu.VMEM((TBLN,), jnp.int32),
            pltpu.VMEM((TBLN,), jnp.int32),
            pltpu.VMEM((REGW,), jnp.float32),
            pltpu.SemaphoreType.DMA,
        ],
    )
    return k(_sortflush_body)(idxf, vr, vi, cc)


NRUN = NW * SUB
RBW = BRUN * REC_W


def _accum_body(brec, offq, lenq, numflat, wts, csq,
                acr, aci, acw, acc, stage, tailb, obuf, lbuf, sem, semt):
    wid = _wid()
    lane = lax.iota(jnp.int32, LANES)
    lane4 = lane * REC_W
    zerof = jnp.zeros((LANES,), jnp.float32)
    onef = jnp.ones((LANES,), jnp.float32)
    QW = QROWS * NKX

    def zero_one(ac):
        def z(i, c2):
            ac[pl.ds(i * 16, 16)] = zerof
            return c2
        lax.fori_loop(0, QW // 16, z, 0)

    for ac0 in (acr, aci, acw, acc):
        zero_one(ac0)

    def rnd(r, carry):
        qs = r * NW + wid
        pltpu.sync_copy(offq.at[qs], obuf)
        pltpu.sync_copy(lenq.at[qs], lbuf)

        def fire(runi, c2):
            off = jnp.max(plsc.load_gather(
                obuf, [jnp.full((LANES,), runi, jnp.int32)]))
            pltpu.async_copy(
                brec.at[pl.ds(pl.multiple_of(off * REC_W, 8), RBW)],
                stage.at[pl.ds(runi * RBW, RBW)], sem)
            return c2

        lax.fori_loop(0, NRUN, fire, 0)

        def drain(runi, c2):
            pltpu.make_async_copy(
                brec.at[pl.ds(0, RBW)],
                stage.at[pl.ds(runi * RBW, RBW)], sem).wait()
            return c2

        lax.fori_loop(0, NRUN, drain, 0)

        def addgrp(buf, bufbase, g, lim):
            m = (g * 16 + lane) < lim
            gi = jnp.full((LANES,), bufbase + g * 16 * REC_W, jnp.int32) + lane4
            rel = plsc.bitcast(plsc.load_gather(buf, [gi]), jnp.int32)
            vrv = plsc.load_gather(buf, [gi + 1])
            viv = plsc.load_gather(buf, [gi + 2])
            ccv = plsc.load_gather(buf, [gi + 3])
            hi = lax.shift_right_logical(rel, 8)
            lo = rel & jnp.int32(255)
            fidx = hi * NKX + lo
            plsc.addupdate_scatter(acr, [fidx], vrv, mask=m)
            plsc.addupdate_scatter(aci, [fidx], viv, mask=m)
            plsc.addupdate_scatter(acw, [fidx], onef, mask=m)
            plsc.addupdate_scatter(acc, [fidx], ccv, mask=m)

        def run_one(runi, c2):
            seglen = jnp.max(plsc.load_gather(
                lbuf, [jnp.full((LANES,), runi, jnp.int32)]))
            off = jnp.max(plsc.load_gather(
                obuf, [jnp.full((LANES,), runi, jnp.int32)]))
            head_n = jnp.minimum(seglen, BRUN)
            nvec = lax.div(head_n + 15, 16)
            limv = jnp.full((LANES,), head_n, jnp.int32)

            def vloop(g, c3):
                addgrp(stage, runi * RBW, g, limv)
                return c3

            lax.fori_loop(0, nvec, vloop, 0)

            nblk = lax.div(seglen + (BRUN - 1), BRUN)

            def tblk(b, c3):
                toff = pl.multiple_of((off + b * BRUN) * REC_W, 8)
                pltpu.sync_copy(brec.at[pl.ds(toff, RBW)], tailb)
                tlim = jnp.full((LANES,), seglen - b * BRUN, jnp.int32)

                def tv(g, c4):
                    addgrp(tailb, 0, g, tlim)
                    return c4

                lax.fori_loop(0, 4, tv, 0)
                return c3

            lax.fori_loop(1, nblk, tblk, 0)
            return c2

        lax.fori_loop(0, NRUN, run_one, 0)

        w0 = qs * QW
        h0 = pltpu.async_copy(acr, numflat.at[pl.ds(w0, QW)], semt)
        h1 = pltpu.async_copy(aci, numflat.at[pl.ds(NZY * NKX + w0, QW)], semt)
        h2 = pltpu.async_copy(acw, wts.at[pl.ds(w0, QW)], semt)
        h3 = pltpu.async_copy(acc, csq.at[pl.ds(w0, QW)], semt)
        for h, ac in ((h0, acr), (h1, aci), (h2, acw), (h3, acc)):
            h.wait()
            zero_one(ac)
        return carry

    lax.fori_loop(0, QS // NW, rnd, 0)


def _accum(brec, offq, lenq):
    k = functools.partial(
        pl.kernel,
        mesh=_mesh(),
        compiler_params=_SC_PARAMS,
        out_type=[
            jax.ShapeDtypeStruct((2 * NZY * NKX,), jnp.float32),
            jax.ShapeDtypeStruct((NZY * NKX,), jnp.float32),
            jax.ShapeDtypeStruct((NZY * NKX,), jnp.float32),
        ],
        scratch_types=[
            pltpu.VMEM((QROWS * NKX,), jnp.float32),
            pltpu.VMEM((QROWS * NKX,), jnp.float32),
            pltpu.VMEM((QROWS * NKX,), jnp.float32),
            pltpu.VMEM((QROWS * NKX,), jnp.float32),
            pltpu.VMEM((NRUN * RBW,), jnp.float32),
            pltpu.VMEM((RBW,), jnp.float32),
            pltpu.VMEM((NRUN,), jnp.int32),
            pltpu.VMEM((NRUN,), jnp.int32),
            pltpu.SemaphoreType.DMA,
            pltpu.SemaphoreType.DMA,
        ],
    )
    return k(_accum_body)(brec, offq, lenq)


def kernel(imgs, ctf, rotMats, hwShiftAngs, numerator, weights, ctfsq):
    f = jnp.fft.rfftn(imgs, axes=(-2, -1))
    fr = jnp.real(f).astype(jnp.float32)
    fi = jnp.imag(f).astype(jnp.float32)
    ky = jnp.fft.fftfreq(D).astype(jnp.float32)
    kx = jnp.fft.rfftfreq(D).astype(jnp.float32)
    syky = hwShiftAngs[:, 0, None] * ky[None, :]
    sxkx = hwShiftAngs[:, 1, None] * kx[None, :]

    yc = (jnp.fft.fftfreq(D) * D).astype(jnp.float32)
    xc = jnp.arange(NKX, dtype=jnp.float32)
    gx = jnp.broadcast_to(xc[None, :], (D, NKX))
    gy = jnp.broadcast_to(yc[:, None], (D, NKX))
    gz = jnp.zeros((D, NKX), dtype=jnp.float32)
    grid = jnp.stack([gx, gy, gz], axis=-1)
    rot = jnp.einsum('bij,hwj->bhwi', rotMats, grid)
    neg = rot[..., 0] < 0
    rot = jnp.where(neg[..., None], -rot, rot)
    xi = jnp.round(rot[..., 0]).astype(jnp.int32)
    yi = jnp.round(rot[..., 1]).astype(jnp.int32)
    zi = jnp.round(rot[..., 2]).astype(jnp.int32)
    half = D // 2
    valid = (xi >= 0) & (xi < NKX) & (jnp.abs(yi) < half) & (jnp.abs(zi) < half)
    yi = jnp.mod(yi, D)
    zi = jnp.mod(zi, D)
    code = (zi * D + yi) * 256 + xi
    idxf = jnp.where(valid, code, SENT).reshape(NPTS)

    vr, vi, cc = _prep(fr, fi, ctf, neg.astype(jnp.float32), syky, sxkx)
    vr = vr.reshape(NPTS)
    vi = vi.reshape(NPTS)
    cc = cc.reshape(NPTS)

    brec, offt, lent = _sortflush(idxf, vr, vi, cc)

    offq = offt[:, :, :QS].reshape(NRUN, QS).T.reshape(QS, NRUN)
    lenq = lent[:, :, :QS].reshape(NRUN, QS).T.reshape(QS, NRUN)

    numflat, wtsf, csqf = _accum(brec, offq, lenq)

    new_num = numflat.reshape(2, D, D, NKX)
    new_w = wtsf.reshape(D, D, NKX)
    new_c = csqf.reshape(D, D, NKX)
    return new_num, new_w, new_c
